# baseline (speedup 1.0000x reference)
import functools

import jax
import jax.numpy as jnp
from jax import lax
from jax.experimental import pallas as pl
from jax.experimental.pallas import tpu as pltpu
from jax.experimental.pallas import tpu_sc as plsc

N = 10000
E = 320000
D = 128

NC = 2
NS = 16
K = 80

NCHUNK_ALL = E // K
NCHUNK_A = NCHUNK_ALL // NS
CPS_A = 5
NSTEP_A = NCHUNK_A // CPS_A
NCHUNK = NCHUNK_ALL // (NC * NS)
NROWBLK = N // 16
ROWS_PER_TILE = N // NS
NG = K // 16


def _finish(partials, W, bias2d):
    def body(p_ref, w_ref, b_ref, o_ref):
        agg = p_ref[0] + p_ref[1]
        o_ref[...] = jnp.dot(agg, w_ref[...],
                             preferred_element_type=jnp.float32) + b_ref[...]

    return pl.pallas_call(
        body,
        grid=(10,),
        in_specs=[
            pl.BlockSpec((2, N // 10, D), lambda j: (0, j, 0)),
            pl.BlockSpec((D, D), lambda j: (0, 0)),
            pl.BlockSpec((1, D), lambda j: (0, 0)),
        ],
        out_specs=pl.BlockSpec((N // 10, D), lambda j: (j, 0)),
        out_shape=jax.ShapeDtypeStruct((N, D), jnp.float32),
    )(partials, W, bias2d)


_sc_mesh = plsc.VectorSubcoreMesh(core_axis_name="c", subcore_axis_name="s")


@functools.partial(
    pl.kernel,
    out_type=jax.ShapeDtypeStruct((NC, N, D), jnp.float32),
    mesh=_sc_mesh,
    scratch_types=[
        pltpu.VMEM((NROWBLK, 16), jnp.float32),
        pltpu.VMEM((3, CPS_A, 3, K), jnp.int32),
        pltpu.VMEM((2, K), jnp.int32),
        pltpu.VMEM((K,), jnp.float32),
        pltpu.VMEM((5, NROWBLK // 5), jnp.int32),
        pltpu.VMEM((K, D), jnp.bfloat16),
        pltpu.VMEM((K, D), jnp.bfloat16),
        pltpu.VMEM((K, D), jnp.bfloat16),
        pltpu.VMEM((K, D), jnp.float32),
        pltpu.VMEM((K, D), jnp.float32),
        pltpu.VMEM_SHARED((NROWBLK, 16), jnp.float32),
        pltpu.VMEM_SHARED((N, D), jnp.float32),
        pltpu.SemaphoreType.DMA,
        pltpu.SemaphoreType.DMA,
        pltpu.SemaphoreType.DMA,
        pltpu.SemaphoreType.DMA,
        pltpu.SemaphoreType.DMA,
        pltpu.SemaphoreType.DMA,
        pltpu.SemaphoreType.DMA,
        pltpu.SemaphoreType.DMA,
        pltpu.SemaphoreType.DMA,
        pltpu.SemaphoreType.DMA,
    ],
    compiler_params=pltpu.CompilerParams(needs_layout_passes=False,
                                         use_tc_tiling_on_sc=False),
)
def _sc_spmm(x, edges3, sidx, out,
             rs_loc, ering, rowv, vals, sidx_v, gbuf0, gbuf1, gbuf2,
             sbuf0, sbuf1, rs_sh, acc_sh,
             esem0, esem1, esem2, gsem0, gsem1, gsem2, ssem0, ssem1, ssem2,
             zsem):
    c = lax.axis_index("c")
    s = lax.axis_index("s")
    gbufs = (gbuf0, gbuf1, gbuf2)
    sbufs = (sbuf0, sbuf1)
    esems = (esem0, esem1, esem2)
    gsems = (gsem0, gsem1, gsem2)
    ssems = (ssem0, ssem1, ssem2)

    pltpu.sync_copy(sidx, sidx_v)
    zv = jnp.zeros((16,), jnp.float32)

    @plsc.parallel_loop(0, K, unroll=4)
    def _(e):
        for f in range(D // 16):
            sbuf0[e, pl.ds(f * 16, 16)] = zv

    def zrs_body(i, carry):
        rs_loc[i, pl.ds(0, 16)] = zv
        return carry

    lax.fori_loop(0, NROWBLK, zrs_body, 0)

    r0 = s * ROWS_PER_TILE
    NZ = ROWS_PER_TILE // K
    RZ = ROWS_PER_TILE - NZ * K
    for i in range(NZ):
        pltpu.async_copy(sbuf0, acc_sh.at[pl.ds(r0 + i * K, K)], zsem)
    pltpu.async_copy(sbuf0.at[pl.ds(0, RZ)],
                     acc_sh.at[pl.ds(r0 + NZ * K, RZ)], zsem)

    @pl.when(s == 0)
    def _():
        pltpu.sync_copy(rs_loc, rs_sh)

    plsc.subcore_barrier()

    qa0 = s * NCHUNK_A

    def acopy(sa, p):
        pltpu.async_copy(edges3.at[pl.ds(qa0 + CPS_A * sa, CPS_A)],
                         ering.at[p], esems[p])

    def await_(p):
        pltpu.make_async_copy(edges3.at[pl.ds(0, CPS_A)], ering.at[p],
                              esems[p]).wait()

    def step_a(sa, p, prefetch):
        await_(p)
        for h in range(CPS_A):
            for g in range(NG):
                sl = pl.ds(g * 16, 16)
                r16 = ering[p, h, 0, sl]
                a16 = plsc.bitcast(ering[p, h, 2, sl], jnp.float32)
                plsc.addupdate_scatter(rs_loc, [r16 >> 4, r16 & 15], a16)
        if prefetch:
            acopy(sa + 3, p)

    for p in range(3):
        acopy(p, p)

    def body_a(a, carry):
        for r in range(3):
            step_a(3 * a + r, r, True)
        return carry

    lax.fori_loop(0, (NSTEP_A - 5) // 3, body_a, 0)
    step_a(NSTEP_A - 5, 0, True)
    step_a(NSTEP_A - 4, 1, True)
    step_a(NSTEP_A - 3, 2, False)
    step_a(NSTEP_A - 2, 0, False)
    step_a(NSTEP_A - 1, 1, False)

    q_len = NROWBLK // 5
    for q in range(5):
        pltpu.sync_copy(rs_loc.at[pl.ds(q * q_len, q_len)],
                        rs_sh.at[sidx_v.at[q]], add=True)

    for i in range(NZ):
        pltpu.make_async_copy(sbuf0, acc_sh.at[pl.ds(r0 + i * K, K)],
                              zsem).wait()
    pltpu.make_async_copy(sbuf0.at[pl.ds(0, RZ)],
                          acc_sh.at[pl.ds(r0 + NZ * K, RZ)], zsem).wait()

    plsc.subcore_barrier()

    pltpu.sync_copy(rs_sh, rs_loc)

    qb0 = (c * NS + s) * NCHUNK

    def ering_copy(q, p):
        pltpu.async_copy(edges3.at[q], ering.at[p, 0], esems[p])

    def ering_wait(p):
        pltpu.make_async_copy(edges3.at[0], ering.at[p, 0], esems[p]).wait()

    def start_gather(pg):
        pltpu.async_copy(x.at[ering.at[pg, 0, 1]], gbufs[pg], gsems[pg])

    def wait_gather(pg):
        pltpu.make_async_copy(x.at[ering.at[pg, 0, 1]], gbufs[pg],
                              gsems[pg]).wait()

    def start_scatter(ps):
        pltpu.async_copy(sbufs[ps], acc_sh.at[rowv.at[ps]], ssems[ps],
                         add=True)

    def wait_scatter(ps):
        pltpu.make_async_copy(sbufs[ps], acc_sh.at[rowv.at[ps]],
                              ssems[ps]).wait()

    def compute_chunk(pg, ps):
        gbuf = gbufs[pg]
        sbuf = sbufs[ps]
        for g in range(NG):
            sl = pl.ds(g * 16, 16)
            c16 = ering[pg, 0, 1, sl]
            a16 = plsc.bitcast(ering[pg, 0, 2, sl], jnp.float32)
            rs16 = plsc.load_gather(rs_loc, [c16 >> 4, c16 & 15])
            vals[sl] = a16 / rs16
            rowv[ps, sl] = ering[pg, 0, 0, sl]

        @plsc.parallel_loop(0, K, unroll=8)
        def _(e):
            e16 = lax.broadcast(e, (16,))
            v16 = plsc.load_gather(vals, [e16])
            for f in range(D // 32):
                m = gbuf[e, pl.ds(f * 32, 32)]
                a, b = plsc.unpack(m, format=plsc.PackFormat.INTERLEAVED)
                sbuf[e, pl.ds(f * 32, 16)] = a * v16
                sbuf[e, pl.ds(f * 32 + 16, 16)] = b * v16

    def step_b(j, pg, ps, wait_sc, do_ering, do_gather):
        wait_gather(pg)
        if wait_sc:
            wait_scatter(ps)
        compute_chunk(pg, ps)
        start_scatter(ps)
        if do_ering:
            ering_copy(qb0 + j + 3, pg)
        if do_gather:
            p2 = (pg + 2) % 3
            ering_wait(p2)
            start_gather(p2)

    for p in range(3):
        ering_copy(qb0 + p, p)
    ering_wait(0)
    start_gather(0)
    ering_wait(1)
    start_gather(1)

    step_b(0, 0, 0, False, True, True)
    step_b(1, 1, 1, False, True, True)

    def body_b(a, carry):
        j0 = 2 + 6 * a
        for u in range(6):
            step_b(j0 + u, (2 + u) % 3, u % 2, True, True, True)
        return carry

    lax.fori_loop(0, (NCHUNK - 5) // 6, body_b, 0)
    step_b(NCHUNK - 3, (NCHUNK - 3) % 3, (NCHUNK - 3) % 2, True, False, True)
    step_b(NCHUNK - 2, (NCHUNK - 2) % 3, (NCHUNK - 2) % 2, True, False, False)
    step_b(NCHUNK - 1, (NCHUNK - 1) % 3, (NCHUNK - 1) % 2, True, False, False)
    wait_scatter((NCHUNK - 2) % 2)
    wait_scatter((NCHUNK - 1) % 2)

    plsc.subcore_barrier()

    pltpu.sync_copy(acc_sh.at[pl.ds(r0, ROWS_PER_TILE)],
                    out.at[c, pl.ds(r0, ROWS_PER_TILE)])


def kernel(layer_in, edge_index, adj_values, W, bias):
    row = edge_index[0]
    col = edge_index[1]
    adj_i = lax.bitcast_convert_type(adj_values, jnp.int32)
    edges3 = jnp.stack(
        [row.reshape(NCHUNK_ALL, K), col.reshape(NCHUNK_ALL, K),
         adj_i.reshape(NCHUNK_ALL, K)], axis=1)
    sidx = jnp.arange(NROWBLK, dtype=jnp.int32).reshape(5, NROWBLK // 5)
    xb = layer_in.astype(jnp.bfloat16).reshape(N, D // 32, 2, 16)
    x2 = xb.transpose(0, 1, 3, 2).reshape(N, D)
    partials = _sc_spmm(x2, edges3, sidx)
    return _finish(partials, W, bias.reshape(1, D))


# speedup vs baseline: 1.0894x; 1.0894x over previous
import functools

import jax
import jax.numpy as jnp
from jax import lax
from jax.experimental import pallas as pl
from jax.experimental.pallas import tpu as pltpu
from jax.experimental.pallas import tpu_sc as plsc

N = 10000
E = 320000
D = 128

NC = 2
NS = 16
K = 80

NCHUNK_ALL = E // K
NCHUNK_A = NCHUNK_ALL // NS
CPS_A = 5
NSTEP_A = NCHUNK_A // CPS_A
NCHUNK = NCHUNK_ALL // (NC * NS)
NROWBLK = N // 16
ROWS_PER_TILE = N // NS
NG = K // 16


def _finish(partials, W, bias2d):
    def body(p_ref, w_ref, b_ref, o_ref):
        agg = p_ref[0] + p_ref[1]
        o_ref[...] = jnp.dot(agg, w_ref[...],
                             preferred_element_type=jnp.float32) + b_ref[...]

    return pl.pallas_call(
        body,
        grid=(10,),
        in_specs=[
            pl.BlockSpec((2, N // 10, D), lambda j: (0, j, 0)),
            pl.BlockSpec((D, D), lambda j: (0, 0)),
            pl.BlockSpec((1, D), lambda j: (0, 0)),
        ],
        out_specs=pl.BlockSpec((N // 10, D), lambda j: (j, 0)),
        out_shape=jax.ShapeDtypeStruct((N, D), jnp.float32),
    )(partials, W, bias2d)


_sc_mesh = plsc.VectorSubcoreMesh(core_axis_name="c", subcore_axis_name="s")


@functools.partial(
    pl.kernel,
    out_type=jax.ShapeDtypeStruct((NC, N, D), jnp.float32),
    mesh=_sc_mesh,
    scratch_types=[
        pltpu.VMEM((NROWBLK, 16), jnp.float32),
        pltpu.VMEM((3, CPS_A, 3, K), jnp.int32),
        pltpu.VMEM((2, K), jnp.int32),
        pltpu.VMEM((K,), jnp.float32),
        pltpu.VMEM((5, NROWBLK // 5), jnp.int32),
        pltpu.VMEM((K, D), jnp.bfloat16),
        pltpu.VMEM((K, D), jnp.bfloat16),
        pltpu.VMEM((K, D), jnp.bfloat16),
        pltpu.VMEM((K, D), jnp.float32),
        pltpu.VMEM((K, D), jnp.float32),
        pltpu.VMEM_SHARED((NROWBLK, 16), jnp.float32),
        pltpu.VMEM_SHARED((N, D), jnp.float32),
        pltpu.SemaphoreType.DMA,
        pltpu.SemaphoreType.DMA,
        pltpu.SemaphoreType.DMA,
        pltpu.SemaphoreType.DMA,
        pltpu.SemaphoreType.DMA,
        pltpu.SemaphoreType.DMA,
        pltpu.SemaphoreType.DMA,
        pltpu.SemaphoreType.DMA,
        pltpu.SemaphoreType.DMA,
        pltpu.SemaphoreType.DMA,
    ],
    compiler_params=pltpu.CompilerParams(needs_layout_passes=False,
                                         use_tc_tiling_on_sc=False),
)
def _sc_spmm(x, edges3, sidx, out,
             rs_loc, ering, rowv, vals, sidx_v, gbuf0, gbuf1, gbuf2,
             sbuf0, sbuf1, rs_sh, acc_sh,
             esem0, esem1, esem2, gsem0, gsem1, gsem2, ssem0, ssem1, ssem2,
             zsem):
    c = lax.axis_index("c")
    s = lax.axis_index("s")
    gbufs = (gbuf0, gbuf1, gbuf2)
    sbufs = (sbuf0, sbuf1)
    esems = (esem0, esem1, esem2)
    gsems = (gsem0, gsem1, gsem2)
    ssems = (ssem0, ssem1, ssem2)

    pltpu.sync_copy(sidx, sidx_v)
    zv = jnp.zeros((16,), jnp.float32)

    @plsc.parallel_loop(0, K, unroll=4)
    def _(e):
        for f in range(D // 16):
            sbuf0[e, pl.ds(f * 16, 16)] = zv

    def zrs_body(i, carry):
        rs_loc[i, pl.ds(0, 16)] = zv
        return carry

    lax.fori_loop(0, NROWBLK, zrs_body, 0)

    r0 = s * ROWS_PER_TILE
    NZ = ROWS_PER_TILE // K
    RZ = ROWS_PER_TILE - NZ * K
    for i in range(NZ):
        pltpu.async_copy(sbuf0, acc_sh.at[pl.ds(r0 + i * K, K)], zsem)
    pltpu.async_copy(sbuf0.at[pl.ds(0, RZ)],
                     acc_sh.at[pl.ds(r0 + NZ * K, RZ)], zsem)

    @pl.when(s == 0)
    def _():
        pltpu.sync_copy(rs_loc, rs_sh)

    plsc.subcore_barrier()

    qa0 = s * NCHUNK_A

    def acopy(sa, p):
        pltpu.async_copy(edges3.at[pl.ds(qa0 + CPS_A * sa, CPS_A)],
                         ering.at[p], esems[p])

    def await_(p):
        pltpu.make_async_copy(edges3.at[pl.ds(0, CPS_A)], ering.at[p],
                              esems[p]).wait()

    def step_a(sa, p, prefetch):
        await_(p)
        for h in range(CPS_A):
            for g in range(NG):
                sl = pl.ds(g * 16, 16)
                r16 = ering[p, h, 0, sl]
                a16 = plsc.bitcast(ering[p, h, 2, sl], jnp.float32)
                plsc.addupdate_scatter(rs_loc, [r16 >> 4, r16 & 15], a16)
        if prefetch:
            acopy(sa + 3, p)

    for p in range(3):
        acopy(p, p)

    def body_a(a, carry):
        for r in range(3):
            step_a(3 * a + r, r, True)
        return carry

    lax.fori_loop(0, (NSTEP_A - 5) // 3, body_a, 0)
    step_a(NSTEP_A - 5, 0, True)
    step_a(NSTEP_A - 4, 1, True)
    step_a(NSTEP_A - 3, 2, False)
    step_a(NSTEP_A - 2, 0, False)
    step_a(NSTEP_A - 1, 1, False)

    q_len = NROWBLK // 5
    for q in range(5):
        pltpu.sync_copy(rs_loc.at[pl.ds(q * q_len, q_len)],
                        rs_sh.at[sidx_v.at[q]], add=True)

    for i in range(NZ):
        pltpu.make_async_copy(sbuf0, acc_sh.at[pl.ds(r0 + i * K, K)],
                              zsem).wait()
    pltpu.make_async_copy(sbuf0.at[pl.ds(0, RZ)],
                          acc_sh.at[pl.ds(r0 + NZ * K, RZ)], zsem).wait()

    plsc.subcore_barrier()

    pltpu.sync_copy(rs_sh, rs_loc)

    qb0 = (c * NS + s) * NCHUNK

    def ering_copy(q, p):
        pltpu.async_copy(edges3.at[q], ering.at[p, 0], esems[p])

    def ering_wait(p):
        pltpu.make_async_copy(edges3.at[0], ering.at[p, 0], esems[p]).wait()

    def start_gather(pg):
        pltpu.async_copy(x.at[ering.at[pg, 0, 1]], gbufs[pg], gsems[pg])

    def wait_gather(pg):
        pltpu.make_async_copy(x.at[ering.at[pg, 0, 1]], gbufs[pg],
                              gsems[pg]).wait()

    def start_scatter(ps):
        pltpu.async_copy(sbufs[ps], acc_sh.at[rowv.at[ps]], ssems[ps],
                         add=True)

    def wait_scatter(ps):
        pltpu.make_async_copy(sbufs[ps], acc_sh.at[rowv.at[ps]],
                              ssems[ps]).wait()

    def compute_chunk(pg, ps):
        gbuf = gbufs[pg]
        sbuf = sbufs[ps]
        for g in range(NG):
            sl = pl.ds(g * 16, 16)
            c16 = ering[pg, 0, 1, sl]
            a16 = plsc.bitcast(ering[pg, 0, 2, sl], jnp.float32)
            rs16 = plsc.load_gather(rs_loc, [c16 >> 4, c16 & 15])
            vals[sl] = a16 / rs16
            rowv[ps, sl] = ering[pg, 0, 0, sl]

        @plsc.parallel_loop(0, K, unroll=4)
        def _(e):
            e16 = lax.broadcast(e, (16,))
            v16 = plsc.load_gather(vals, [e16])
            for f in range(D // 32):
                m = gbuf[e, pl.ds(f * 32, 32)]
                a, b = plsc.unpack(m, format=plsc.PackFormat.INTERLEAVED)
                sbuf[e, pl.ds(f * 32, 16)] = a * v16
                sbuf[e, pl.ds(f * 32 + 16, 16)] = b * v16

    def step_b(j, pg, ps, wait_sc, do_ering, do_gather):
        wait_gather(pg)
        if wait_sc:
            wait_scatter(ps)
        compute_chunk(pg, ps)
        start_scatter(ps)
        if do_ering:
            ering_copy(qb0 + j + 3, pg)
        if do_gather:
            p2 = (pg + 2) % 3
            ering_wait(p2)
            start_gather(p2)

    for p in range(3):
        ering_copy(qb0 + p, p)
    ering_wait(0)
    start_gather(0)
    ering_wait(1)
    start_gather(1)

    step_b(0, 0, 0, False, True, True)
    step_b(1, 1, 1, False, True, True)

    def body_b(a, carry):
        j0 = 2 + 6 * a
        for u in range(6):
            step_b(j0 + u, (2 + u) % 3, u % 2, True, True, True)
        return carry

    lax.fori_loop(0, (NCHUNK - 5) // 6, body_b, 0)
    step_b(NCHUNK - 3, (NCHUNK - 3) % 3, (NCHUNK - 3) % 2, True, False, True)
    step_b(NCHUNK - 2, (NCHUNK - 2) % 3, (NCHUNK - 2) % 2, True, False, False)
    step_b(NCHUNK - 1, (NCHUNK - 1) % 3, (NCHUNK - 1) % 2, True, False, False)
    wait_scatter((NCHUNK - 2) % 2)
    wait_scatter((NCHUNK - 1) % 2)

    plsc.subcore_barrier()

    pltpu.sync_copy(acc_sh.at[pl.ds(r0, ROWS_PER_TILE)],
                    out.at[c, pl.ds(r0, ROWS_PER_TILE)])


def kernel(layer_in, edge_index, adj_values, W, bias):
    row = edge_index[0]
    col = edge_index[1]
    adj_i = lax.bitcast_convert_type(adj_values, jnp.int32)
    edges3 = jnp.stack(
        [row.reshape(NCHUNK_ALL, K), col.reshape(NCHUNK_ALL, K),
         adj_i.reshape(NCHUNK_ALL, K)], axis=1)
    sidx = jnp.arange(NROWBLK, dtype=jnp.int32).reshape(5, NROWBLK // 5)
    x2 = layer_in.astype(jnp.bfloat16)
    blk = jnp.arange(0, D, 32)[:, None]
    perm = (blk + jnp.concatenate([jnp.arange(0, 32, 2),
                                   jnp.arange(1, 32, 2)])).reshape(D)
    partials = _sc_spmm(x2, edges3, sidx)
    return _finish(partials, W[perm], bias.reshape(1, D))


# speedup vs baseline: 1.1091x; 1.0181x over previous
import functools

import jax
import jax.numpy as jnp
from jax import lax
from jax.experimental import pallas as pl
from jax.experimental.pallas import tpu as pltpu
from jax.experimental.pallas import tpu_sc as plsc

N = 10000
E = 320000
D = 128

NC = 2
NS = 16
K = 80

NCHUNK_ALL = E // K
NCHUNK_A = NCHUNK_ALL // NS
CPS_A = 5
NSTEP_A = NCHUNK_A // CPS_A
NCHUNK = NCHUNK_ALL // (NC * NS)
NROWBLK = N // 16
ROWS_PER_TILE = N // NS
NG = K // 16


def _finish(partials, W, bias2d):
    def body(p_ref, w_ref, b_ref, o_ref):
        agg = p_ref[0] + p_ref[1]
        o_ref[...] = jnp.dot(agg, w_ref[...],
                             preferred_element_type=jnp.float32) + b_ref[...]

    return pl.pallas_call(
        body,
        grid=(10,),
        in_specs=[
            pl.BlockSpec((2, N // 10, D), lambda j: (0, j, 0)),
            pl.BlockSpec((D, D), lambda j: (0, 0)),
            pl.BlockSpec((1, D), lambda j: (0, 0)),
        ],
        out_specs=pl.BlockSpec((N // 10, D), lambda j: (j, 0)),
        out_shape=jax.ShapeDtypeStruct((N, D), jnp.float32),
    )(partials, W, bias2d)


_sc_mesh = plsc.VectorSubcoreMesh(core_axis_name="c", subcore_axis_name="s")


@functools.partial(
    pl.kernel,
    out_type=jax.ShapeDtypeStruct((NC, N, D), jnp.float32),
    mesh=_sc_mesh,
    scratch_types=[
        pltpu.VMEM((NROWBLK, 16), jnp.float32),
        pltpu.VMEM((3, CPS_A, 3, K), jnp.int32),
        pltpu.VMEM((2, K), jnp.int32),
        pltpu.VMEM((K,), jnp.float32),
        pltpu.VMEM((5, NROWBLK // 5), jnp.int32),
        pltpu.VMEM((K, D), jnp.bfloat16),
        pltpu.VMEM((K, D), jnp.bfloat16),
        pltpu.VMEM((K, D), jnp.bfloat16),
        pltpu.VMEM((K, D), jnp.float32),
        pltpu.VMEM((K, D), jnp.float32),
        pltpu.VMEM_SHARED((NROWBLK, 16), jnp.float32),
        pltpu.VMEM_SHARED((N, D), jnp.float32),
        pltpu.SemaphoreType.DMA,
        pltpu.SemaphoreType.DMA,
        pltpu.SemaphoreType.DMA,
        pltpu.SemaphoreType.DMA,
        pltpu.SemaphoreType.DMA,
        pltpu.SemaphoreType.DMA,
        pltpu.SemaphoreType.DMA,
        pltpu.SemaphoreType.DMA,
        pltpu.SemaphoreType.DMA,
        pltpu.SemaphoreType.DMA,
    ],
    compiler_params=pltpu.CompilerParams(needs_layout_passes=False,
                                         use_tc_tiling_on_sc=False),
)
def _sc_spmm(x, edges3, sidx, out,
             rs_loc, ering, rowv, vals, sidx_v, gbuf0, gbuf1, gbuf2,
             sbuf0, sbuf1, rs_sh, acc_sh,
             esem0, esem1, esem2, gsem0, gsem1, gsem2, ssem0, ssem1, ssem2,
             zsem):
    c = lax.axis_index("c")
    s = lax.axis_index("s")
    gbufs = (gbuf0, gbuf1, gbuf2)
    sbufs = (sbuf0, sbuf1)
    esems = (esem0, esem1, esem2)
    gsems = (gsem0, gsem1, gsem2)
    ssems = (ssem0, ssem1, ssem2)

    pltpu.sync_copy(sidx, sidx_v)
    zv = jnp.zeros((16,), jnp.float32)

    @plsc.parallel_loop(0, K, unroll=4)
    def _(e):
        for f in range(D // 16):
            sbuf0[e, pl.ds(f * 16, 16)] = zv

    def zrs_body(i, carry):
        rs_loc[i, pl.ds(0, 16)] = zv
        return carry

    lax.fori_loop(0, NROWBLK, zrs_body, 0)

    r0 = s * ROWS_PER_TILE
    NZ = ROWS_PER_TILE // K
    RZ = ROWS_PER_TILE - NZ * K
    for i in range(NZ):
        pltpu.async_copy(sbuf0, acc_sh.at[pl.ds(r0 + i * K, K)], zsem)
    pltpu.async_copy(sbuf0.at[pl.ds(0, RZ)],
                     acc_sh.at[pl.ds(r0 + NZ * K, RZ)], zsem)

    @pl.when(s == 0)
    def _():
        pltpu.sync_copy(rs_loc, rs_sh)

    plsc.subcore_barrier()

    qa0 = s * NCHUNK_A

    def acopy(sa, p):
        pltpu.async_copy(edges3.at[pl.ds(qa0 + CPS_A * sa, CPS_A)],
                         ering.at[p], esems[p])

    def await_(p):
        pltpu.make_async_copy(edges3.at[pl.ds(0, CPS_A)], ering.at[p],
                              esems[p]).wait()

    def step_a(sa, p, prefetch):
        await_(p)
        for h in range(CPS_A):
            for g in range(NG):
                sl = pl.ds(g * 16, 16)
                r16 = ering[p, h, 0, sl]
                a16 = plsc.bitcast(ering[p, h, 2, sl], jnp.float32)
                plsc.addupdate_scatter(rs_loc, [r16 >> 4, r16 & 15], a16)
        if prefetch:
            acopy(sa + 3, p)

    for p in range(3):
        acopy(p, p)

    def body_a(a, carry):
        for r in range(3):
            step_a(3 * a + r, r, True)
        return carry

    lax.fori_loop(0, (NSTEP_A - 5) // 3, body_a, 0)
    step_a(NSTEP_A - 5, 0, True)
    step_a(NSTEP_A - 4, 1, True)
    step_a(NSTEP_A - 3, 2, False)
    step_a(NSTEP_A - 2, 0, False)
    step_a(NSTEP_A - 1, 1, False)

    q_len = NROWBLK // 5
    for q in range(5):
        pltpu.sync_copy(rs_loc.at[pl.ds(q * q_len, q_len)],
                        rs_sh.at[sidx_v.at[q]], add=True)

    for i in range(NZ):
        pltpu.make_async_copy(sbuf0, acc_sh.at[pl.ds(r0 + i * K, K)],
                              zsem).wait()
    pltpu.make_async_copy(sbuf0.at[pl.ds(0, RZ)],
                          acc_sh.at[pl.ds(r0 + NZ * K, RZ)], zsem).wait()

    plsc.subcore_barrier()

    pltpu.sync_copy(rs_sh, rs_loc)

    qb0 = (c * NS + s) * NCHUNK

    def ering_copy(q, p):
        pltpu.async_copy(edges3.at[q], ering.at[p, 0], esems[p])

    def ering_wait(p):
        pltpu.make_async_copy(edges3.at[0], ering.at[p, 0], esems[p]).wait()

    def start_gather(pg):
        pltpu.async_copy(x.at[ering.at[pg, 0, 1]], gbufs[pg], gsems[pg])

    def wait_gather(pg):
        pltpu.make_async_copy(x.at[ering.at[pg, 0, 1]], gbufs[pg],
                              gsems[pg]).wait()

    def start_scatter(ps):
        pltpu.async_copy(sbufs[ps], acc_sh.at[rowv.at[ps]], ssems[ps],
                         add=True)

    def wait_scatter(ps):
        pltpu.make_async_copy(sbufs[ps], acc_sh.at[rowv.at[ps]],
                              ssems[ps]).wait()

    def vals_part(pg, ps):
        for g in range(NG):
            sl = pl.ds(g * 16, 16)
            c16 = ering[pg, 0, 1, sl]
            a16 = plsc.bitcast(ering[pg, 0, 2, sl], jnp.float32)
            rs16 = plsc.load_gather(rs_loc, [c16 >> 4, c16 & 15])
            vals[sl] = a16 / rs16
            rowv[ps, sl] = ering[pg, 0, 0, sl]

    def scale_part(pg, ps):
        gbuf = gbufs[pg]
        sbuf = sbufs[ps]

        @plsc.parallel_loop(0, K, unroll=4)
        def _(e):
            e16 = lax.broadcast(e, (16,))
            v16 = plsc.load_gather(vals, [e16])
            for f in range(D // 32):
                m = gbuf[e, pl.ds(f * 32, 32)]
                a, b = plsc.unpack(m, format=plsc.PackFormat.INTERLEAVED)
                sbuf[e, pl.ds(f * 32, 16)] = a * v16
                sbuf[e, pl.ds(f * 32 + 16, 16)] = b * v16

    def step_b(j, pg, ps, wait_sc, do_ering, do_gather):
        wait_gather(pg)
        if wait_sc:
            wait_scatter(ps)
        vals_part(pg, ps)
        if do_ering:
            ering_copy(qb0 + j + 3, pg)
        if do_gather:
            p2 = (pg + 2) % 3
            ering_wait(p2)
            start_gather(p2)
        scale_part(pg, ps)
        start_scatter(ps)

    for p in range(3):
        ering_copy(qb0 + p, p)
    ering_wait(0)
    start_gather(0)
    ering_wait(1)
    start_gather(1)

    step_b(0, 0, 0, False, True, True)
    step_b(1, 1, 1, False, True, True)

    def body_b(a, carry):
        j0 = 2 + 6 * a
        for u in range(6):
            step_b(j0 + u, (2 + u) % 3, u % 2, True, True, True)
        return carry

    lax.fori_loop(0, (NCHUNK - 5) // 6, body_b, 0)
    step_b(NCHUNK - 3, (NCHUNK - 3) % 3, (NCHUNK - 3) % 2, True, False, True)
    step_b(NCHUNK - 2, (NCHUNK - 2) % 3, (NCHUNK - 2) % 2, True, False, False)
    step_b(NCHUNK - 1, (NCHUNK - 1) % 3, (NCHUNK - 1) % 2, True, False, False)
    wait_scatter((NCHUNK - 2) % 2)
    wait_scatter((NCHUNK - 1) % 2)

    plsc.subcore_barrier()

    pltpu.sync_copy(acc_sh.at[pl.ds(r0, ROWS_PER_TILE)],
                    out.at[c, pl.ds(r0, ROWS_PER_TILE)])


def kernel(layer_in, edge_index, adj_values, W, bias):
    row = edge_index[0]
    col = edge_index[1]
    adj_i = lax.bitcast_convert_type(adj_values, jnp.int32)
    edges3 = jnp.stack(
        [row.reshape(NCHUNK_ALL, K), col.reshape(NCHUNK_ALL, K),
         adj_i.reshape(NCHUNK_ALL, K)], axis=1)
    sidx = jnp.arange(NROWBLK, dtype=jnp.int32).reshape(5, NROWBLK // 5)
    x2 = layer_in.astype(jnp.bfloat16)
    blk = jnp.arange(0, D, 32)[:, None]
    perm = (blk + jnp.concatenate([jnp.arange(0, 32, 2),
                                   jnp.arange(1, 32, 2)])).reshape(D)
    partials = _sc_spmm(x2, edges3, sidx)
    return _finish(partials, W[perm], bias.reshape(1, D))
